# Initial kernel scaffold; baseline (speedup 1.0000x reference)
import jax
import jax.numpy as jnp
from jax.experimental import pallas as pl


def kernel(user, item, user_embedding, item_embedding, mgnn_weight, W, item_propagate_W, item_behavior_W, rel_rows, rel_cols, rel_vals, ig_rows, ig_cols, ig_vals, tm_rows, tm_cols, tm_vals, user_behavior_degree, item_graph_degree):
    raise NotImplementedError("write your pallas kernel here")


# speedup vs baseline: 17.1110x; 17.1110x over previous
import functools

import jax
import jax.numpy as jnp
from jax import lax
from jax.experimental import pallas as pl
from jax.experimental.pallas import tpu as pltpu
from jax.experimental.pallas import tpu_sc as plsc

NC = 2
NS = 16
NW = NC * NS
LN = 16

LAMB = 0.5
L2_NORM = 1e-4
F32 = jnp.float32
I32 = jnp.int32


def _iota16(base):
    return lax.iota(I32, LN) + base


def _make_stage_a(I, D, EG, E):
    BK = 128
    assert EG % BK == 0 and E % BK == 0
    mesh = plsc.VectorSubcoreMesh(core_axis_name="c", subcore_axis_name="s")
    RSUB = (I // NS) & ~15

    @functools.partial(
        pl.kernel,
        out_type=jax.ShapeDtypeStruct((NC, 4, I, D), F32),
        mesh=mesh,
        compiler_params=pltpu.CompilerParams(use_tc_tiling_on_sc=False, needs_layout_passes=False),
        scratch_types=[
            pltpu.VMEM_SHARED((4, I, D), F32),
            pltpu.VMEM((BK,), I32),
            pltpu.VMEM((BK,), I32),
            pltpu.VMEM((BK, 32), F32),
            pltpu.SemaphoreType.DMA,
        ],
    )
    def stage_a(ig_rows, ig_cols, tm_rows, tm_cols, item_emb, user_emb,
                out, acc, cidx, ridx, gbuf, sem):
        cid = lax.axis_index("c")
        sid = lax.axis_index("s")
        w = sid * NC + cid

        def _zrow(j, _):
            gbuf[j, pl.ds(0, LN)] = jnp.zeros((LN,), F32)
            gbuf[j, pl.ds(LN, LN)] = jnp.zeros((LN,), F32)
            return 0
        lax.fori_loop(0, BK, _zrow, 0)
        my_rows = jnp.where(sid == NS - 1, I - (NS - 1) * RSUB, RSUB)
        my_base = sid * RSUB
        for k in range(4):
            def _zacc(q, _, k=k):
                pltpu.sync_copy(gbuf.at[pl.ds(0, LN)],
                                acc.at[k, pl.ds(my_base + q * LN, LN), :])
                return 0
            lax.fori_loop(0, my_rows // LN, _zacc, 0)
        plsc.subcore_barrier()

        def run_job(k, rows_hbm, cols_hbm, src_hbm, nedges, off):
            tot = nedges // BK
            nblk = (tot - w + NW - 1) // NW

            def body(t, _):
                base = off + (w + t * NW) * BK
                pltpu.sync_copy(cols_hbm.at[pl.ds(base, BK)], cidx)
                pltpu.async_copy(src_hbm.at[cidx], gbuf, sem).wait()
                pltpu.sync_copy(rows_hbm.at[pl.ds(base, BK)], ridx)
                pltpu.sync_copy(gbuf, acc.at[k].at[ridx], add=True)
                return 0
            lax.fori_loop(0, nblk, body, 0)

        for k in range(3):
            run_job(k, ig_rows, ig_cols, item_emb, EG, k * EG)
        run_job(3, tm_cols, tm_rows, user_emb, E, 0)
        plsc.subcore_barrier()

        for k in range(4):
            def _wout(q, _, k=k):
                pltpu.sync_copy(acc.at[k, pl.ds(my_base + q * LN, LN), :],
                                out.at[cid, k, pl.ds(my_base + q * LN, LN), :])
                return 0
            lax.fori_loop(0, my_rows // LN, _wout, 0)

    return stage_a


def _make_stage_b(I, D, R):
    WC = 80
    TB = 2000
    assert I % TB == 0

    def body(part_ref, iemb_ref, igdeg_ref, ipw_ref, w_ref, g_ref, c_ref):
        p = part_ref[0] + part_ref[1]
        iemb = iemb_ref[...]
        itemf = jnp.dot(p[3], w_ref[...], preferred_element_type=F32)
        g_ref[:, 0:D] = itemf
        g_ref[:, D:2 * D] = iemb
        ones = jnp.ones((TB, 1), F32)
        zeros = jnp.zeros((TB, WC - 2 * D - 1), F32)
        for i in range(R):
            tip2 = jnp.dot(p[i] / (igdeg_ref[i] + 1e-8), ipw_ref[i],
                           preferred_element_type=F32)
            g_ref[:, (2 + i) * D:(3 + i) * D] = tip2
            c_ref[i, :, 0:D] = iemb
            c_ref[i, :, D:2 * D] = tip2
            c_ref[i, :, 2 * D:2 * D + 1] = ones
            c_ref[i, :, 2 * D + 1:WC] = zeros

    return pl.pallas_call(
        body,
        grid=(I // TB,),
        in_specs=[
            pl.BlockSpec((NC, 4, TB, D), lambda i: (0, 0, i, 0)),
            pl.BlockSpec((TB, D), lambda i: (i, 0)),
            pl.BlockSpec((R, TB, 1), lambda i: (0, i, 0)),
            pl.BlockSpec((R, D, D), lambda i: (0, 0, 0)),
            pl.BlockSpec((D, D), lambda i: (0, 0)),
        ],
        out_specs=(pl.BlockSpec((TB, 5 * D), lambda i: (i, 0)),
                   pl.BlockSpec((R, TB, WC), lambda i: (0, i, 0))),
        out_shape=(jax.ShapeDtypeStruct((I, 5 * D), F32),
                   jax.ShapeDtypeStruct((R, I, WC), F32)),
    ), WC


def _make_stage_c(U, I, D, R, E, B, BL, WC):
    LUT_CH = 3136
    LUTP = NS * LUT_CH
    assert LUTP >= U
    BP = 1040
    TRASH = B
    EB = 3200
    assert E % EB == 0
    TOTB = E // EB
    NIN = EB // LN
    GGC = 80
    ROWS_W = BL // NW
    assert ROWS_W % GGC == 0
    UB_S = B // NS

    mesh = plsc.VectorSubcoreMesh(core_axis_name="c", subcore_axis_name="s")

    @functools.partial(
        pl.kernel,
        out_type=(
            jax.ShapeDtypeStruct((NC, R, B, WC), F32),
            jax.ShapeDtypeStruct((BL, 5 * D), F32),
            jax.ShapeDtypeStruct((B, D), F32),
            jax.ShapeDtypeStruct((NC, R, BP, WC), F32),
        ),
        mesh=mesh,
        compiler_params=pltpu.CompilerParams(use_tc_tiling_on_sc=False, needs_layout_passes=False),
        scratch_types=[
            pltpu.VMEM_SHARED((LUTP,), I32),
            pltpu.VMEM_SHARED((R, BP, WC), F32),
            pltpu.VMEM((LUTP,), I32),
            pltpu.VMEM((LUT_CH,), I32),
            pltpu.VMEM((8, 128), I32),
            pltpu.VMEM((8, 128), I32),
            pltpu.VMEM((EB,), I32),
            pltpu.VMEM((EB,), I32),
            pltpu.VMEM((32,), I32),
            pltpu.VMEM((32,), I32),
            pltpu.VMEM((LN, WC), F32),
            pltpu.VMEM((LN, WC), F32),
            pltpu.VMEM((GGC,), I32),
            pltpu.VMEM((GGC, 5 * D), F32),
            pltpu.VMEM((UB_S,), I32),
            pltpu.VMEM((UB_S, D), F32),
            pltpu.SemaphoreType.DMA,
        ],
    )
    def stage_c(user, itemf, rel_rows, rel_cols, c_tab, g_tab, user_emb,
                baccs, gg, ueb, accraw,
                lut_sp, acc, lutv, ibuf, ubuf, vbuf, rbuf, cbuf,
                colsP, slotsP, g16, zb, gidx, gbuf, myu, uebuf, sem):
        cid = lax.axis_index("c")
        sid = lax.axis_index("s")
        w = sid * NC + cid

        def _fill(t, _):
            ibuf[pl.ds(t * LN, LN)] = jnp.full((LN,), TRASH, I32)
            return 0
        lax.fori_loop(0, LUT_CH // LN, _fill, 0)
        pltpu.sync_copy(ibuf, lut_sp.at[pl.ds(sid * LUT_CH, LUT_CH)])

        def _zrow(j, _):
            for q in range(WC // LN):
                zb[j, pl.ds(q * LN, LN)] = jnp.zeros((LN,), F32)
            return 0
        lax.fori_loop(0, LN, _zrow, 0)
        arows = jnp.where(sid == NS - 1, BP - (NS - 1) * 64, 64)
        abase = sid * 64
        for i in range(R):
            def _zacc(q, _, i=i):
                pltpu.sync_copy(zb, acc.at[i, pl.ds(abase + q * LN, LN), :])
                return 0
            lax.fori_loop(0, arows // LN, _zacc, 0)
        plsc.subcore_barrier()

        @pl.when(sid == 0)
        def _scatter_users():
            def _j(j, _):
                pltpu.sync_copy(user.at[pl.ds(j * 128, 128)], ubuf.at[j])
                def _g(g, _):
                    vbuf[j, pl.ds(g * LN, LN)] = _iota16(j * 128 + g * LN)
                    return 0
                lax.fori_loop(0, 8, _g, 0)
                pltpu.sync_copy(vbuf.at[j], lut_sp.at[ubuf.at[j]])
                return 0
            lax.fori_loop(0, 8, _j, 0)
        plsc.subcore_barrier()

        pltpu.sync_copy(lut_sp, lutv)

        for i in range(R):
            c_i = c_tab.at[i]
            nblk = (TOTB - w + NW - 1) // NW

            def blk_body(t, _, i=i, c_i=c_i):
                base = i * E + (w + t * NW) * EB
                pltpu.sync_copy(rel_rows.at[pl.ds(base, EB)], rbuf)
                pltpu.sync_copy(rel_cols.at[pl.ds(base, EB)], cbuf)

                def inner(j, n):
                    r16 = rbuf[pl.ds(j * LN, LN)]
                    c16 = cbuf[pl.ds(j * LN, LN)]
                    s16 = plsc.load_gather(lutv, [r16])
                    m = s16 < TRASH
                    plsc.store_compressed(colsP.at[pl.ds(n, LN)], c16, mask=m)
                    plsc.store_compressed(slotsP.at[pl.ds(n, LN)], s16, mask=m)
                    n = n + jnp.sum(m.astype(I32))

                    @pl.when(n >= LN)
                    def _drain():
                        cols16 = colsP[pl.ds(n - LN, LN)]
                        slots16 = slotsP[pl.ds(n - LN, LN)]
                        pltpu.async_copy(c_i.at[cols16], g16, sem).wait()
                        pltpu.sync_copy(g16, acc.at[i].at[slots16], add=True)
                    return jnp.where(n >= LN, n - LN, n)

                n = lax.fori_loop(0, NIN, inner, jnp.int32(0))

                @pl.when(n > 0)
                def _flush():
                    colsP[pl.ds(n, LN)] = jnp.zeros((LN,), I32)
                    slotsP[pl.ds(n, LN)] = jnp.full((LN,), TRASH, I32)
                    cols16 = colsP[pl.ds(0, LN)]
                    slots16 = slotsP[pl.ds(0, LN)]
                    pltpu.async_copy(c_i.at[cols16], g16, sem).wait()
                    pltpu.sync_copy(g16, acc.at[i].at[slots16], add=True)
                return 0

            lax.fori_loop(0, nblk, blk_body, 0)
        plsc.subcore_barrier()

        for i in range(R):
            def _wacc(q, _, i=i):
                pltpu.sync_copy(acc.at[i, pl.ds(abase + q * LN, LN), :],
                                accraw.at[cid, i,
                                          pl.ds(abase + q * LN, LN), :])
                return 0
            lax.fori_loop(0, arows // LN, _wacc, 0)
        plsc.subcore_barrier()

        pltpu.sync_copy(user.at[pl.ds(sid * UB_S, UB_S)], myu)
        for g in range(UB_S // LN):
            u16 = myu[pl.ds(g * LN, LN)]
            s16 = plsc.load_gather(lutv, [u16])
            for i in range(R):
                pltpu.async_copy(accraw.at[cid, i].at[s16], g16, sem).wait()
                pltpu.sync_copy(
                    g16, baccs.at[cid, i, pl.ds(sid * UB_S + g * LN, LN), :])

        def gg_body(t, _):
            base = w * ROWS_W + t * GGC
            pltpu.sync_copy(itemf.at[pl.ds(base, GGC)], gidx)
            pltpu.async_copy(g_tab.at[gidx], gbuf, sem).wait()
            pltpu.sync_copy(gbuf, gg.at[pl.ds(base, GGC), :])
            return 0
        lax.fori_loop(0, ROWS_W // GGC, gg_body, 0)

        @pl.when(cid == 0)
        def _ue():
            pltpu.async_copy(user_emb.at[myu], uebuf, sem).wait()
            pltpu.sync_copy(uebuf, ueb.at[pl.ds(sid * UB_S, UB_S), :])

    return stage_c, BP


def _make_stage_d(B, L, D, R, WC):
    TB = 256
    assert B % TB == 0

    def body(baccs_ref, gg_ref, ueb_ref, mg_ref, ibw_ref, sc_ref, l2_ref):
        acc = baccs_ref[0] + baccs_ref[1]
        mg = mg_ref[...]
        ueb = ueb_ref[...]
        deg = acc[:, :, 2 * D]
        tot = sum(deg[i] * mg[i] for i in range(R))
        ufp = jnp.zeros((TB, D), F32)
        tus = []
        for i in range(R):
            denom = deg[i][:, None] + 1e-8
            nn = acc[i, :, :2 * D] / denom
            ubw = (deg[i] * mg[i]) / (tot + 1e-8)
            ufp = ufp + ubw[:, None] * nn[:, :D]
            tus.append(jnp.dot(nn, ibw_ref[i], preferred_element_type=F32))
        s = LAMB / R
        v = jnp.concatenate(
            [ufp, ueb + s * sum(t[:, :D] for t in tus)]
            + [s * tus[i][:, D:2 * D] for i in range(R)], axis=1)
        gg = gg_ref[...].reshape(TB, L, 5 * D)
        sc_ref[...] = jnp.sum(v[:, None, :] * gg, axis=2)
        l2 = L2_NORM * (L * (jnp.sum(ufp * ufp) + jnp.sum(ueb * ueb))
                        + jnp.sum(gg[:, :, :2 * D] * gg[:, :, :2 * D]))

        @pl.when(pl.program_id(0) == 0)
        def _init():
            l2_ref[...] = jnp.zeros((1, 1), F32)
        l2_ref[...] += jnp.reshape(l2, (1, 1))

    return pl.pallas_call(
        body,
        grid=(B // TB,),
        in_specs=[
            pl.BlockSpec((NC, R, TB, WC), lambda i: (0, 0, i, 0)),
            pl.BlockSpec((TB * L, 5 * D), lambda i: (i, 0)),
            pl.BlockSpec((TB, D), lambda i: (i, 0)),
            pl.BlockSpec((R,), lambda i: (0,)),
            pl.BlockSpec((R, 2 * D, 2 * D), lambda i: (0, 0, 0)),
        ],
        out_specs=(pl.BlockSpec((TB, L), lambda i: (i, 0)),
                   pl.BlockSpec((1, 1), lambda i: (0, 0))),
        out_shape=(jax.ShapeDtypeStruct((B, L), F32),
                   jax.ShapeDtypeStruct((1, 1), F32)),
    )


def kernel(user, item, user_embedding, item_embedding, mgnn_weight, W,
           item_propagate_W, item_behavior_W,
           rel_rows, rel_cols, rel_vals, ig_rows, ig_cols, ig_vals,
           tm_rows, tm_cols, tm_vals, user_behavior_degree, item_graph_degree):
    U, D = user_embedding.shape
    I = item_embedding.shape[0]
    R = mgnn_weight.shape[0]
    B, L = item.shape
    E = rel_rows.shape[1]
    EG = ig_rows.shape[1]
    BL = B * L

    stage_a = _make_stage_a(I, D, EG, E)
    partials = stage_a(ig_rows.reshape(-1), ig_cols.reshape(-1),
                       tm_rows, tm_cols, item_embedding, user_embedding)

    stage_b, WC = _make_stage_b(I, D, R)
    g_tab, c_tab = stage_b(partials, item_embedding, item_graph_degree,
                           item_propagate_W, W)

    stage_c, BP = _make_stage_c(U, I, D, R, E, B, BL, WC)
    baccs, gg, ueb, _ = stage_c(user, item.reshape(BL), rel_rows.reshape(-1),
                                rel_cols.reshape(-1), c_tab, g_tab,
                                user_embedding)

    stage_d = _make_stage_d(B, L, D, R, WC)
    scores, l2 = stage_d(baccs, gg, ueb, mgnn_weight, item_behavior_W)
    return scores, l2[0, 0]
